# baseline (speedup 1.0000x reference)
import functools

import jax
import jax.numpy as jnp
from jax import lax
from jax.experimental import pallas as pl
from jax.experimental.pallas import tpu as pltpu
from jax.experimental.pallas import tpu_sc as plsc

B = 4
C = 512
RAW = 10242
NEW = RAW * 4 - 6
DOWN = NEW - RAW
ROWS = B * C
L = 16

NUM_WORKERS = 32
ROWS_PER_WORKER = ROWS // NUM_WORKERS
CHALF = C // 2

TOP_ITERS = RAW // L
TOP_LAST = RAW - L
DOWN_ITERS = DOWN // L

TOP_BLOCKS = -(-RAW // (2 * L))
DOWN_BLOCKS = DOWN // (2 * L)


def _row_ids(half, t16, k):
    u = t16 * ROWS_PER_WORKER + k
    b = u // CHALF
    c_local = u % CHALF
    rout = b * C + half * CHALF + c_local
    rp0 = b * C + 2 * c_local
    return rout, rp0


RAWP = (TOP_ITERS + 1) * L


def _upsample_body(x_hbm, top_hbm, d_hbm, out_hbm,
                   top_v, d_v, self_v, b_v, m0_v, m1_v, out_v,
                   sem_self, sem_m, sem_out):
    cid = lax.axis_index("c")
    sid = lax.axis_index("s")
    wid = sid * 2 + cid
    half = wid // 16
    t16 = wid % 16


    def issue_pair(kc, mref):
        _, rp0 = _row_ids(half, t16, kc)
        pltpu.async_copy(x_hbm.at[rp0], mref, sem_m)
        pltpu.async_copy(x_hbm.at[rp0 + 1], b_v, sem_m)

    def add_pair(mref):
        @plsc.parallel_loop(0, TOP_ITERS * L, L, unroll=8)
        def add_it(off):
            plsc.addupdate(mref.at[pl.ds(off, L)], b_v[pl.ds(off, L)])

        lane = lax.iota(jnp.int32, L)
        tail = jnp.where(lane >= L - 2, b_v[pl.ds(TOP_LAST, L)], 0.0)
        mref[pl.ds(TOP_LAST, L)] = mref[pl.ds(TOP_LAST, L)] + tail

    def issue_self(k):
        rout, _ = _row_ids(half, t16, k)
        pltpu.async_copy(x_hbm.at[rout], self_v, sem_self)

    def wait_pair():
        pltpu.make_async_copy(x_hbm.at[0], b_v, sem_m).wait()
        pltpu.make_async_copy(x_hbm.at[0], b_v, sem_m).wait()

    def wait_self():
        pltpu.make_async_copy(x_hbm.at[0], self_v, sem_self).wait()

    def drain_out():
        pltpu.make_async_copy(out_hbm.at[0], out_v, sem_out).wait()

    pltpu.sync_copy(top_hbm, top_v)
    pltpu.sync_copy(d_hbm.at[half], d_v)
    issue_self(0)
    issue_pair(0, m0_v)
    wait_pair()
    add_pair(m0_v)

    def do_row(k, carry):
        rout, _ = _row_ids(half, t16, k)
        knext = jnp.minimum(k + 1, ROWS_PER_WORKER - 1)
        wait_self()

        @pl.when(k > 0)
        def _():
            drain_out()

        @plsc.parallel_loop(0, TOP_BLOCKS * L, L, unroll=8)
        def top_it(w):
            word = top_v[pl.ds(w, L)]
            lo = word & 0xFFFF
            hi = word >> 16
            out_v[pl.ds(w * 2, L)] = plsc.load_gather(self_v, [lo])
            out_v[pl.ds(w * 2 + L, L)] = plsc.load_gather(self_v, [hi])

        issue_self(knext)
        even = k % 2 == 0

        @pl.when(even)
        def _():
            issue_pair(knext, m1_v)

        @pl.when(jnp.logical_not(even))
        def _():
            issue_pair(knext, m0_v)

        def down_loop(mref):
            @plsc.parallel_loop(0, DOWN_BLOCKS * L, L, unroll=8)
            def down_it(w):
                word = d_v[pl.ds(w, L)]
                lo = word & 0xFFFF
                hi = word >> 16
                out_v[pl.ds(RAW + w * 2, L)] = (
                    plsc.load_gather(mref, [lo]) * 0.5)
                out_v[pl.ds(RAW + w * 2 + L, L)] = (
                    plsc.load_gather(mref, [hi]) * 0.5)

        @pl.when(even)
        def _():
            down_loop(m0_v)

        @pl.when(jnp.logical_not(even))
        def _():
            down_loop(m1_v)

        wait_pair()

        @pl.when(even)
        def _():
            add_pair(m1_v)

        @pl.when(jnp.logical_not(even))
        def _():
            add_pair(m0_v)

        pltpu.async_copy(out_v, out_hbm.at[rout], sem_out)
        return carry

    lax.fori_loop(0, ROWS_PER_WORKER, do_row, 0)
    drain_out()
    wait_self()


@jax.jit
def _upsample(x2d, top_src, d2):
    mesh = plsc.VectorSubcoreMesh(core_axis_name="c", subcore_axis_name="s")
    run = functools.partial(
        pl.kernel,
        mesh=mesh,
        compiler_params=pltpu.CompilerParams(needs_layout_passes=False),
        out_type=jax.ShapeDtypeStruct((ROWS, NEW), jnp.float32),
        scratch_types=[
            pltpu.VMEM((TOP_BLOCKS * L,), jnp.int32),
            pltpu.VMEM((DOWN_BLOCKS * L,), jnp.int32),
            pltpu.VMEM((RAW,), jnp.float32),
            pltpu.VMEM((RAW,), jnp.float32),
            pltpu.VMEM((RAW,), jnp.float32),
            pltpu.VMEM((RAW,), jnp.float32),
            pltpu.VMEM((NEW,), jnp.float32),
            pltpu.SemaphoreType.DMA,
            pltpu.SemaphoreType.DMA,
            pltpu.SemaphoreType.DMA,
        ],
    )(_upsample_body)
    return run(x2d, top_src, d2)


def _pack_pairs(idx):
    t = idx.reshape(-1, 2, L)
    return (t[:, 0] | (t[:, 1] << 16)).reshape(-1)


def kernel(x, upconv_top_index, upconv_down_index):
    x2d = x.reshape(ROWS, RAW)
    top_src = (upconv_top_index // 7).astype(jnp.int32)
    top_p = _pack_pairs(jnp.pad(top_src, (0, TOP_BLOCKS * 2 * L - RAW)))
    d = upconv_down_index.reshape(-1, 2)
    d0 = (d[:, 0] // 7).astype(jnp.int32)
    d1 = (d[:, 1] // 7).astype(jnp.int32)
    d2 = jnp.stack([_pack_pairs(d0), _pack_pairs(d1)])
    out = _upsample(x2d, top_p, d2)
    return out.reshape(B, C, NEW)


# speedup vs baseline: 1.0803x; 1.0803x over previous
import functools

import jax
import jax.numpy as jnp
from jax import lax
from jax.experimental import pallas as pl
from jax.experimental.pallas import tpu as pltpu
from jax.experimental.pallas import tpu_sc as plsc

B = 4
C = 512
RAW = 10242
NEW = RAW * 4 - 6
DOWN = NEW - RAW
ROWS = B * C
L = 16

NUM_WORKERS = 32
ROWS_PER_WORKER = ROWS // NUM_WORKERS
CHALF = C // 2

TOP_ITERS = RAW // L
TOP_LAST = RAW - L
DOWN_ITERS = DOWN // L

TOP_BLOCKS = -(-RAW // (2 * L))
DOWN_BLOCKS = DOWN // (2 * L)


def _row_ids(half, t16, k):
    u = t16 * ROWS_PER_WORKER + k
    b = u // CHALF
    c_local = u % CHALF
    rout = b * C + half * CHALF + c_local
    rp0 = b * C + 2 * c_local
    return rout, rp0


def _upsample_body(x_hbm, top_hbm, d_hbm, out_hbm,
                   top_v, d_v, self_v, a_v, b_v, m_v, out_v,
                   sem_in, sem_out):
    cid = lax.axis_index("c")
    sid = lax.axis_index("s")
    wid = sid * 2 + cid
    half = wid // 16
    t16 = wid % 16

    def issue_in(k):
        rout, rp0 = _row_ids(half, t16, k)
        pltpu.async_copy(x_hbm.at[rout], self_v, sem_in)
        pltpu.async_copy(x_hbm.at[rp0], a_v, sem_in)
        pltpu.async_copy(x_hbm.at[rp0 + 1], b_v, sem_in)

    def wait_in():
        pltpu.make_async_copy(x_hbm.at[0], self_v, sem_in).wait()
        pltpu.make_async_copy(x_hbm.at[0], a_v, sem_in).wait()
        pltpu.make_async_copy(x_hbm.at[0], b_v, sem_in).wait()

    def drain_out():
        pltpu.make_async_copy(out_hbm.at[0], out_v, sem_out).wait()

    pltpu.sync_copy(top_hbm, top_v)
    pltpu.sync_copy(d_hbm.at[half], d_v)
    issue_in(0)

    def do_row(k, carry):
        rout, _ = _row_ids(half, t16, k)
        wait_in()

        @plsc.parallel_loop(0, TOP_ITERS * L, L, unroll=8)
        def mean_it(off):
            m_v[pl.ds(off, L)] = (a_v[pl.ds(off, L)] + b_v[pl.ds(off, L)]) * 0.5

        m_v[pl.ds(TOP_LAST, L)] = (
            a_v[pl.ds(TOP_LAST, L)] + b_v[pl.ds(TOP_LAST, L)]) * 0.5

        @pl.when(k > 0)
        def _():
            drain_out()

        @plsc.parallel_loop(0, TOP_BLOCKS * L, L, unroll=8)
        def top_it(w):
            word = top_v[pl.ds(w, L)]
            lo = word & 0xFFFF
            hi = word >> 16
            out_v[pl.ds(w * 2, L)] = plsc.load_gather(self_v, [lo])
            out_v[pl.ds(w * 2 + L, L)] = plsc.load_gather(self_v, [hi])

        issue_in(jnp.minimum(k + 1, ROWS_PER_WORKER - 1))

        @plsc.parallel_loop(0, DOWN_BLOCKS * L, L, unroll=16)
        def down_it(w):
            word = d_v[pl.ds(w, L)]
            lo = word & 0xFFFF
            hi = word >> 16
            out_v[pl.ds(RAW + w * 2, L)] = plsc.load_gather(m_v, [lo])
            out_v[pl.ds(RAW + w * 2 + L, L)] = plsc.load_gather(m_v, [hi])

        pltpu.async_copy(out_v, out_hbm.at[rout], sem_out)
        return carry

    lax.fori_loop(0, ROWS_PER_WORKER, do_row, 0)
    drain_out()
    wait_in()


@jax.jit
def _upsample(x2d, top_src, d2):
    mesh = plsc.VectorSubcoreMesh(core_axis_name="c", subcore_axis_name="s")
    run = functools.partial(
        pl.kernel,
        mesh=mesh,
        compiler_params=pltpu.CompilerParams(needs_layout_passes=False),
        out_type=jax.ShapeDtypeStruct((ROWS, NEW), jnp.float32),
        scratch_types=[
            pltpu.VMEM((TOP_BLOCKS * L,), jnp.int32),
            pltpu.VMEM((DOWN_BLOCKS * L,), jnp.int32),
            pltpu.VMEM((RAW,), jnp.float32),
            pltpu.VMEM((RAW,), jnp.float32),
            pltpu.VMEM((RAW,), jnp.float32),
            pltpu.VMEM((RAW,), jnp.float32),
            pltpu.VMEM((NEW,), jnp.float32),
            pltpu.SemaphoreType.DMA,
            pltpu.SemaphoreType.DMA,
        ],
    )(_upsample_body)
    return run(x2d, top_src, d2)


def _pack_pairs(idx):
    t = idx.reshape(-1, 2, L)
    return (t[:, 0] | (t[:, 1] << 16)).reshape(-1)


def kernel(x, upconv_top_index, upconv_down_index):
    x2d = x.reshape(ROWS, RAW)
    top_src = (upconv_top_index // 7).astype(jnp.int32)
    top_p = _pack_pairs(jnp.pad(top_src, (0, TOP_BLOCKS * 2 * L - RAW)))
    d = upconv_down_index.reshape(-1, 2)
    d0 = (d[:, 0] // 7).astype(jnp.int32)
    d1 = (d[:, 1] // 7).astype(jnp.int32)
    d2 = jnp.stack([_pack_pairs(d0), _pack_pairs(d1)])
    out = _upsample(x2d, top_p, d2)
    return out.reshape(B, C, NEW)


# speedup vs baseline: 1.0811x; 1.0008x over previous
import functools

import jax
import jax.numpy as jnp
from jax import lax
from jax.experimental import pallas as pl
from jax.experimental.pallas import tpu as pltpu
from jax.experimental.pallas import tpu_sc as plsc

B = 4
C = 512
RAW = 10242
NEW = RAW * 4 - 6
DOWN = NEW - RAW
ROWS = B * C
L = 16

NUM_WORKERS = 32
ROWS_PER_WORKER = ROWS // NUM_WORKERS
CHALF = C // 2

TOP_ITERS = RAW // L
TOP_LAST = RAW - L
DOWN_ITERS = DOWN // L

TOP_BLOCKS = -(-RAW // (2 * L))
DOWN_BLOCKS = DOWN // (2 * L)


def _row_ids(half, t16, k):
    u = t16 * ROWS_PER_WORKER + k
    b = u // CHALF
    c_local = u % CHALF
    rout = b * C + half * CHALF + c_local
    rp0 = b * C + 2 * c_local
    return rout, rp0


def _upsample_body(x_hbm, top_hbm, d_hbm, out_hbm,
                   top_v, d_v, self_v, a_v, b_v, m_v, out_v,
                   sem_self, sem_ab, sem_out):
    cid = lax.axis_index("c")
    sid = lax.axis_index("s")
    wid = sid * 2 + cid
    half = wid // 16
    t16 = wid % 16

    def issue_self(k):
        rout, _ = _row_ids(half, t16, k)
        pltpu.async_copy(x_hbm.at[rout], self_v, sem_self)

    def issue_ab(k):
        _, rp0 = _row_ids(half, t16, k)
        pltpu.async_copy(x_hbm.at[rp0], a_v, sem_ab)
        pltpu.async_copy(x_hbm.at[rp0 + 1], b_v, sem_ab)

    def wait_self():
        pltpu.make_async_copy(x_hbm.at[0], self_v, sem_self).wait()

    def wait_ab():
        pltpu.make_async_copy(x_hbm.at[0], a_v, sem_ab).wait()
        pltpu.make_async_copy(x_hbm.at[0], b_v, sem_ab).wait()

    def drain_out():
        pltpu.make_async_copy(out_hbm.at[0], out_v, sem_out).wait()

    pltpu.sync_copy(top_hbm, top_v)
    pltpu.sync_copy(d_hbm.at[half], d_v)
    issue_self(0)
    issue_ab(0)

    def do_row(k, carry):
        rout, _ = _row_ids(half, t16, k)
        knext = jnp.minimum(k + 1, ROWS_PER_WORKER - 1)
        wait_ab()

        @plsc.parallel_loop(0, TOP_ITERS * L, L, unroll=8)
        def mean_it(off):
            m_v[pl.ds(off, L)] = (a_v[pl.ds(off, L)] + b_v[pl.ds(off, L)]) * 0.5

        m_v[pl.ds(TOP_LAST, L)] = (
            a_v[pl.ds(TOP_LAST, L)] + b_v[pl.ds(TOP_LAST, L)]) * 0.5

        issue_ab(knext)
        wait_self()

        @pl.when(k > 0)
        def _():
            drain_out()

        @plsc.parallel_loop(0, TOP_BLOCKS * L, L, unroll=8)
        def top_it(w):
            word = top_v[pl.ds(w, L)]
            lo = word & 0xFFFF
            hi = word >> 16
            out_v[pl.ds(w * 2, L)] = plsc.load_gather(self_v, [lo])
            out_v[pl.ds(w * 2 + L, L)] = plsc.load_gather(self_v, [hi])

        issue_self(knext)

        @plsc.parallel_loop(0, DOWN_BLOCKS * L, L, unroll=16)
        def down_it(w):
            word = d_v[pl.ds(w, L)]
            lo = word & 0xFFFF
            hi = word >> 16
            out_v[pl.ds(RAW + w * 2, L)] = plsc.load_gather(m_v, [lo])
            out_v[pl.ds(RAW + w * 2 + L, L)] = plsc.load_gather(m_v, [hi])

        pltpu.async_copy(out_v, out_hbm.at[rout], sem_out)
        return carry

    lax.fori_loop(0, ROWS_PER_WORKER, do_row, 0)
    drain_out()
    wait_self()
    wait_ab()


@jax.jit
def _upsample(x2d, top_src, d2):
    mesh = plsc.VectorSubcoreMesh(core_axis_name="c", subcore_axis_name="s")
    run = functools.partial(
        pl.kernel,
        mesh=mesh,
        compiler_params=pltpu.CompilerParams(needs_layout_passes=False),
        out_type=jax.ShapeDtypeStruct((ROWS, NEW), jnp.float32),
        scratch_types=[
            pltpu.VMEM((TOP_BLOCKS * L,), jnp.int32),
            pltpu.VMEM((DOWN_BLOCKS * L,), jnp.int32),
            pltpu.VMEM((RAW,), jnp.float32),
            pltpu.VMEM((RAW,), jnp.float32),
            pltpu.VMEM((RAW,), jnp.float32),
            pltpu.VMEM((RAW,), jnp.float32),
            pltpu.VMEM((NEW,), jnp.float32),
            pltpu.SemaphoreType.DMA,
            pltpu.SemaphoreType.DMA,
            pltpu.SemaphoreType.DMA,
        ],
    )(_upsample_body)
    return run(x2d, top_src, d2)


def _pack_pairs(idx):
    t = idx.reshape(-1, 2, L)
    return (t[:, 0] | (t[:, 1] << 16)).reshape(-1)


def kernel(x, upconv_top_index, upconv_down_index):
    x2d = x.reshape(ROWS, RAW)
    top_src = (upconv_top_index // 7).astype(jnp.int32)
    top_p = _pack_pairs(jnp.pad(top_src, (0, TOP_BLOCKS * 2 * L - RAW)))
    d = upconv_down_index.reshape(-1, 2)
    d0 = (d[:, 0] // 7).astype(jnp.int32)
    d1 = (d[:, 1] // 7).astype(jnp.int32)
    d2 = jnp.stack([_pack_pairs(d0), _pack_pairs(d1)])
    out = _upsample(x2d, top_p, d2)
    return out.reshape(B, C, NEW)
